# baseline (speedup 1.0000x reference)
import jax
import jax.numpy as jnp
from jax.experimental import pallas as pl
from jax.experimental.pallas import tpu as pltpu

_NUM_SAMPLE = 2048
_G = 128
_C = 128


def _fps_kernel(pts_ref, outx_ref, outy_ref, outz_ref, dists_ref):
    _, NBLK, B, C = pts_ref.shape
    S = outx_ref.shape[0] * _G

    lane = jax.lax.broadcasted_iota(jnp.int32, (B, _G), 1)
    flane = jax.lax.broadcasted_iota(jnp.int32, (B, C), 1).astype(jnp.float32)

    dists_ref[...] = jnp.full((NBLK, B, C), 1e10, dtype=jnp.float32)

    fx0 = pts_ref[0, 0, :, 0:1]
    fy0 = pts_ref[1, 0, :, 0:1]
    fz0 = pts_ref[2, 0, :, 0:1]
    buf0 = jnp.zeros((B, _G), dtype=jnp.float32)

    neg = jnp.full((B, C), -1e30, dtype=jnp.float32)
    zero = jnp.zeros((B, C), dtype=jnp.float32)

    def body(i, carry):
        fx, fy, fz, bx, by, bz = carry
        col = jax.lax.rem(i, _G)
        g = jax.lax.div(i, _G)
        hit = lane == col
        bx = jnp.where(hit, fx, bx)
        by = jnp.where(hit, fy, by)
        bz = jnp.where(hit, fz, bz)
        outx_ref[pl.ds(g, 1)] = bx.reshape(1, B, _G)
        outy_ref[pl.ds(g, 1)] = by.reshape(1, B, _G)
        outz_ref[pl.ds(g, 1)] = bz.reshape(1, B, _G)

        def update(j, acc):
            amax, aj, ax, ay, az = acc
            xc = pts_ref[0, j]
            yc = pts_ref[1, j]
            zc = pts_ref[2, j]
            dx = xc - fx
            dy = yc - fy
            dz = zc - fz
            d = (dx * dx + dz * dz) + dy * dy
            nd = jnp.minimum(dists_ref[j], d)
            dists_ref[j] = nd
            cmp = nd > amax
            jf = j.astype(jnp.float32)
            amax = jnp.where(cmp, nd, amax)
            aj = jnp.where(cmp, jf, aj)
            ax = jnp.where(cmp, xc, ax)
            ay = jnp.where(cmp, yc, ay)
            az = jnp.where(cmp, zc, az)
            return (amax, aj, ax, ay, az)

        def chunk2(t, acc):
            accA, accB = acc
            accA = update(2 * t, accA)
            accB = update(2 * t + 1, accB)
            return (accA, accB)

        acc0 = (neg, zero, zero, zero, zero)
        accA, accB = jax.lax.fori_loop(
            0, NBLK // 2, chunk2, (acc0, acc0), unroll=4)

        amaxA, ajA, axA, ayA, azA = accA
        amaxB, ajB, axB, ayB, azB = accB
        anA = jnp.float32(C) * ajA + flane
        anB = jnp.float32(C) * ajB + flane
        takeB = (amaxB > amaxA) | ((amaxB == amaxA) & (anB < anA))
        amax = jnp.where(takeB, amaxB, amaxA)
        an = jnp.where(takeB, anB, anA)
        ax = jnp.where(takeB, axB, axA)
        ay = jnp.where(takeB, ayB, ayA)
        az = jnp.where(takeB, azB, azA)

        m = jnp.max(amax, axis=1, keepdims=True)
        big = jnp.float32(NBLK * C)
        cand = jnp.where(amax == m, an, big)
        nstar = jnp.min(cand, axis=1, keepdims=True)
        oh = an == nstar
        nfx = jnp.max(jnp.where(oh, ax, -1e30), axis=1, keepdims=True)
        nfy = jnp.max(jnp.where(oh, ay, -1e30), axis=1, keepdims=True)
        nfz = jnp.max(jnp.where(oh, az, -1e30), axis=1, keepdims=True)
        return (nfx, nfy, nfz, bx, by, bz)

    jax.lax.fori_loop(0, S, body, (fx0, fy0, fz0, buf0, buf0, buf0),
                      unroll=False)


def _run(points):
    B, N, _ = points.shape
    S = _NUM_SAMPLE
    nblk = N // _C
    pts = points.transpose(2, 0, 1).reshape(3, B, nblk, _C)
    pts = pts.transpose(0, 2, 1, 3)

    plane = jax.ShapeDtypeStruct((S // _G, B, _G), jnp.float32)
    return pl.pallas_call(
        _fps_kernel,
        out_shape=(plane, plane, plane),
        scratch_shapes=[pltpu.VMEM((nblk, B, _C), jnp.float32)],
    )(pts)


def kernel(points):
    B, _, _ = points.shape
    S = _NUM_SAMPLE
    ox, oy, oz = _run(points)
    samples = jnp.stack([ox, oy, oz], axis=-1)
    return samples.transpose(1, 0, 2, 3).reshape(B, S, 3)


# speedup vs baseline: 1.2586x; 1.2586x over previous
import jax
import jax.numpy as jnp
from jax.experimental import pallas as pl
from jax.experimental.pallas import tpu as pltpu

_NUM_SAMPLE = 2048
_G = 128
_C = 128


def _fps_kernel(pts_ref, outx_ref, outy_ref, outz_ref, dists_ref):
    _, NBLK, B, C = pts_ref.shape
    S = outx_ref.shape[0] * _G

    lane = jax.lax.broadcasted_iota(jnp.int32, (B, _G), 1)

    dists_ref[...] = jnp.full((NBLK, B, C), 1e10, dtype=jnp.float32)

    fx0 = pts_ref[0, 0, :, 0:1]
    fy0 = pts_ref[1, 0, :, 0:1]
    fz0 = pts_ref[2, 0, :, 0:1]
    buf0 = jnp.zeros((B, _G), dtype=jnp.float32)

    neg = jnp.full((B, C), -1e30, dtype=jnp.float32)
    zero = jnp.zeros((B, C), dtype=jnp.float32)

    def body(i, carry):
        fx, fy, fz, bx, by, bz = carry
        col = jax.lax.rem(i, _G)
        g = jax.lax.div(i, _G)
        hit = lane == col
        bx = jnp.where(hit, fx, bx)
        by = jnp.where(hit, fy, by)
        bz = jnp.where(hit, fz, bz)
        outx_ref[pl.ds(g, 1)] = bx.reshape(1, B, _G)
        outy_ref[pl.ds(g, 1)] = by.reshape(1, B, _G)
        outz_ref[pl.ds(g, 1)] = bz.reshape(1, B, _G)

        def chunk(j, acc):
            amax, ax, ay, az = acc
            xc = pts_ref[0, j]
            yc = pts_ref[1, j]
            zc = pts_ref[2, j]
            dx = xc - fx
            dy = yc - fy
            dz = zc - fz
            d = (dx * dx + dz * dz) + dy * dy
            nd = jnp.minimum(dists_ref[j], d)
            dists_ref[j] = nd
            cmp = nd > amax
            amax = jnp.where(cmp, nd, amax)
            ax = jnp.where(cmp, xc, ax)
            ay = jnp.where(cmp, yc, ay)
            az = jnp.where(cmp, zc, az)
            return (amax, ax, ay, az)

        amax, ax, ay, az = jax.lax.fori_loop(
            0, NBLK, chunk, (neg, zero, zero, zero), unroll=8)

        lstar = jnp.argmax(amax, axis=1).astype(jnp.int32)[:, None]
        oh = lane == lstar
        nfx = jnp.max(jnp.where(oh, ax, -1e30), axis=1, keepdims=True)
        nfy = jnp.max(jnp.where(oh, ay, -1e30), axis=1, keepdims=True)
        nfz = jnp.max(jnp.where(oh, az, -1e30), axis=1, keepdims=True)
        return (nfx, nfy, nfz, bx, by, bz)

    jax.lax.fori_loop(0, S, body, (fx0, fy0, fz0, buf0, buf0, buf0),
                      unroll=False)


def _run(points):
    B, N, _ = points.shape
    S = _NUM_SAMPLE
    nblk = N // _C
    pts = points.transpose(2, 0, 1).reshape(3, B, _C, nblk)
    pts = pts.transpose(0, 3, 1, 2)

    plane = jax.ShapeDtypeStruct((S // _G, B, _G), jnp.float32)
    return pl.pallas_call(
        _fps_kernel,
        out_shape=(plane, plane, plane),
        scratch_shapes=[pltpu.VMEM((nblk, B, _C), jnp.float32)],
    )(pts)


def kernel(points):
    B, _, _ = points.shape
    S = _NUM_SAMPLE
    ox, oy, oz = _run(points)
    samples = jnp.stack([ox, oy, oz], axis=-1)
    return samples.transpose(1, 0, 2, 3).reshape(B, S, 3)
